# Initial kernel scaffold; baseline (speedup 1.0000x reference)
import jax
import jax.numpy as jnp
from jax.experimental import pallas as pl


def kernel(user_profiles, edge_attrs, edge_index):
    raise NotImplementedError("write your pallas kernel here")


# speedup vs baseline: 29.6355x; 29.6355x over previous
import jax
import jax.numpy as jnp
from jax import lax
from jax.experimental import pallas as pl
from jax.experimental.pallas import tpu as pltpu
from jax.experimental.pallas import tpu_sc as plsc

N = 10000
E = 320000
B = 128
NC = 2
NS = 16
L = 16

EP = 327680
NPAD = 10240
NTRASH = 224

STAT_E = EP // NS
PROP_E = EP // (NC * NS)
BLK = 2048
CHUNK = 128
CPB = BLK // CHUNK
SROW = NPAD // B


def _sqrt16(y):
    i = plsc.bitcast(y, jnp.int32)
    r = plsc.bitcast(jnp.int32(0x5F3759DF) - (i >> 1), jnp.float32)
    for _ in range(3):
        r = r * (1.5 - 0.5 * y * r * r)
    return y * r


def _sc_body(from_hbm, attr_hbm, to2d_hbm, upT_hbm, out_hbm,
             fromb, attrb, to2d, s_in_loc, s_out_loc, riota, wbuf, rows,
             s_in_sh, s_out_sh, scores_sh):
    c = lax.axis_index("c")
    s = lax.axis_index("s")

    def zero_srow(r, carry):
        for g in range(B // L):
            s_in_loc[r, pl.ds(g * L, L)] = jnp.zeros((L,), jnp.float32)
            s_out_loc[r, pl.ds(g * L, L)] = jnp.zeros((L,), jnp.float32)
        return carry
    lax.fori_loop(0, SROW, zero_srow, 0)

    def zero_rows(r, carry):
        for g in range(B // L):
            rows[r, pl.ds(g * L, L)] = jnp.zeros((L,), jnp.float32)
        return carry
    lax.fori_loop(0, CHUNK, zero_rows, 0)

    for r in range(SROW // L):
        riota[pl.ds(r * L, L)] = lax.iota(jnp.int32, L) + r * L

    def stat_block(blk, carry):
        base = pl.multiple_of(s * STAT_E + blk * BLK, BLK)
        rowb = pl.multiple_of(s * (STAT_E // CHUNK) + blk * CPB, 8)
        pltpu.sync_copy(to2d_hbm.at[pl.ds(rowb, CPB)], to2d)
        pltpu.sync_copy(from_hbm.at[pl.ds(base, BLK)], fromb)
        pltpu.sync_copy(attr_hbm.at[pl.ds(base, BLK)], attrb)

        def grp(g, carry2):
            r = g // (B // L)
            k = g - r * (B // L)
            ex = jnp.exp(attrb[pl.ds(g * L, L)])
            tt = to2d[r, pl.ds(k * L, L)]
            ff = fromb[pl.ds(g * L, L)]
            plsc.addupdate_scatter(s_in_loc, [tt >> 7, tt & 127], ex)
            plsc.addupdate_scatter(s_out_loc, [ff >> 7, ff & 127], ex)
            return carry2
        lax.fori_loop(0, BLK // L, grp, 0)
        return carry
    lax.fori_loop(0, STAT_E // BLK, stat_block, 0)

    for k in range(NPAD // NS // CHUNK):
        pltpu.sync_copy(rows, scores_sh.at[pl.ds(pl.multiple_of(s * (NPAD // NS) + k * CHUNK, 8), CHUNK)])

    @pl.when(s == 0)
    def _():
        pltpu.sync_copy(s_in_loc, s_in_sh)
        pltpu.sync_copy(s_out_loc, s_out_sh)
    plsc.subcore_barrier()

    @pl.when(s != 0)
    def _():
        pltpu.sync_copy(s_in_loc, s_in_sh.at[riota], add=True)
        pltpu.sync_copy(s_out_loc, s_out_sh.at[riota], add=True)
    plsc.subcore_barrier()

    pltpu.sync_copy(s_in_sh, s_in_loc)
    pltpu.sync_copy(s_out_sh, s_out_loc)

    def prop_block(blk, carry):
        base = pl.multiple_of(c * (EP // NC) + s * PROP_E + blk * BLK, BLK)
        rowb = pl.multiple_of(c * (EP // NC // CHUNK) + s * (PROP_E // CHUNK)
                              + blk * CPB, 8)
        pltpu.sync_copy(to2d_hbm.at[pl.ds(rowb, CPB)], to2d)
        pltpu.sync_copy(from_hbm.at[pl.ds(base, BLK)], fromb)
        pltpu.sync_copy(attr_hbm.at[pl.ds(base, BLK)], attrb)

        def chunk_body(jj, carry2):
            for k in range(CHUNK // L):
                sl = pl.ds(jj * CHUNK + k * L, L)
                ex = jnp.exp(attrb[sl])
                tt = to2d[jj, pl.ds(k * L, L)]
                ff = fromb[sl]
                g_in = plsc.load_gather(s_in_loc, [tt >> 7, tt & 127])
                g_out = plsc.load_gather(s_out_loc, [ff >> 7, ff & 127])
                y = (ex * ex) / (g_in * g_out) + 1e-10
                wbuf[pl.ds(k * L, L)] = _sqrt16(y)

            pltpu.sync_copy(upT_hbm.at[fromb.at[pl.ds(jj * CHUNK, CHUNK)]], rows)

            def scale_edge(e, carry3):
                wv = plsc.load_gather(wbuf, [jnp.full((L,), e, jnp.int32)])
                for v in range(B // L):
                    sl = pl.ds(v * L, L)
                    rows[e, sl] = rows[e, sl] * wv
                return carry3
            lax.fori_loop(0, CHUNK, scale_edge, 0)

            pltpu.sync_copy(rows, scores_sh.at[to2d.at[jj]], add=True)
            return carry2
        lax.fori_loop(0, CPB, chunk_body, 0)
        return carry
    lax.fori_loop(0, PROP_E // BLK, prop_block, 0)

    plsc.subcore_barrier()

    @pl.when(s < 10)
    def _():
        r0 = pl.multiple_of(s * 1000, 8)
        pltpu.sync_copy(scores_sh.at[pl.ds(r0, 1000)],
                        out_hbm.at[c, pl.ds(r0, 1000)])


@jax.jit
def _sc_spmm(from_, attrs, to2d, upT):
    mesh = plsc.VectorSubcoreMesh(core_axis_name="c", subcore_axis_name="s",
                                  num_cores=NC, num_subcores=NS)
    return pl.kernel(
        _sc_body,
        out_type=jax.ShapeDtypeStruct((NC, N, B), jnp.float32),
        mesh=mesh,
        compiler_params=pltpu.CompilerParams(needs_layout_passes=False),
        scratch_types=[
            pltpu.VMEM((BLK,), jnp.int32),
            pltpu.VMEM((BLK,), jnp.float32),
            pltpu.VMEM((CPB, CHUNK), jnp.int32),
            pltpu.VMEM((SROW, B), jnp.float32),
            pltpu.VMEM((SROW, B), jnp.float32),
            pltpu.VMEM((SROW,), jnp.int32),
            pltpu.VMEM((CHUNK,), jnp.float32),
            pltpu.VMEM((CHUNK, B), jnp.float32),
            pltpu.VMEM_SHARED((SROW, B), jnp.float32),
            pltpu.VMEM_SHARED((SROW, B), jnp.float32),
            pltpu.VMEM_SHARED((NPAD, B), jnp.float32),
        ],
    )(from_, attrs, to2d, upT)


def _combine_body(p_ref, o_ref):
    o_ref[...] = (p_ref[0] + p_ref[1]).T


@jax.jit
def _tc_combine(partials):
    return pl.pallas_call(
        _combine_body,
        out_shape=jax.ShapeDtypeStruct((B, N), jnp.float32),
    )(partials)


def kernel(user_profiles, edge_attrs, edge_index):
    npad = EP - E
    pad_idx = N + jnp.arange(npad, dtype=jnp.int32) % NTRASH
    from_ = jnp.concatenate([edge_index[0], pad_idx])
    to_ = jnp.concatenate([edge_index[1], pad_idx])
    attrs = jnp.concatenate([edge_attrs, jnp.zeros((npad,), jnp.float32)])
    to2d = to_.reshape(EP // CHUNK, CHUNK)
    upT = jnp.pad(user_profiles.T, ((0, NPAD - N), (0, 0)))
    partials = _sc_spmm(from_, attrs, to2d, upT)
    return _tc_combine(partials)


# speedup vs baseline: 38.4786x; 1.2984x over previous
import jax
import jax.numpy as jnp
from jax import lax
from jax.experimental import pallas as pl
from jax.experimental.pallas import tpu as pltpu
from jax.experimental.pallas import tpu_sc as plsc

N = 10000
E = 320000
B = 128
NC = 2
NS = 16
L = 16

EP = 327680
NPAD = 10240
NTRASH = 224

STAT_E = EP // NS
PROP_E = EP // (NC * NS)
BLK = 2048
CHUNK = 64
CPB = BLK // CHUNK
SROW = NPAD // B


def _sqrt16(y):
    i = plsc.bitcast(y, jnp.int32)
    r = plsc.bitcast(jnp.int32(0x5F3759DF) - (i >> 1), jnp.float32)
    for _ in range(3):
        r = r * (1.5 - 0.5 * y * r * r)
    return y * r


def _sc_body(from_hbm, attr_hbm, to2d_hbm, upT_hbm, out_hbm,
             fromb, attrb, to2d, s_in_loc, s_out_loc, riota, wbuf,
             rows0, rows1, gsem0, gsem1,
             s_in_sh, s_out_sh, scores_sh):
    c = lax.axis_index("c")
    s = lax.axis_index("s")

    def zero_srow(r, carry):
        for g in range(B // L):
            s_in_loc[r, pl.ds(g * L, L)] = jnp.zeros((L,), jnp.float32)
            s_out_loc[r, pl.ds(g * L, L)] = jnp.zeros((L,), jnp.float32)
        return carry
    lax.fori_loop(0, SROW, zero_srow, 0)

    def zero_rows(r, carry):
        for g in range(B // L):
            rows0[r, pl.ds(g * L, L)] = jnp.zeros((L,), jnp.float32)
        return carry
    lax.fori_loop(0, CHUNK, zero_rows, 0)

    for r in range(SROW // L):
        riota[pl.ds(r * L, L)] = lax.iota(jnp.int32, L) + r * L

    def stat_block(blk, carry):
        base = pl.multiple_of(s * STAT_E + blk * BLK, BLK)
        rowb = pl.multiple_of(s * (STAT_E // CHUNK) + blk * CPB, 8)
        pltpu.sync_copy(to2d_hbm.at[pl.ds(rowb, CPB)], to2d)
        pltpu.sync_copy(from_hbm.at[pl.ds(base, BLK)], fromb)
        pltpu.sync_copy(attr_hbm.at[pl.ds(base, BLK)], attrb)

        def grp(g, carry2):
            r = g // (CHUNK // L)
            k = g - r * (CHUNK // L)
            ex = jnp.exp(attrb[pl.ds(g * L, L)])
            tt = to2d[r, pl.ds(k * L, L)]
            ff = fromb[pl.ds(g * L, L)]
            plsc.addupdate_scatter(s_in_loc, [tt >> 7, tt & 127], ex)
            plsc.addupdate_scatter(s_out_loc, [ff >> 7, ff & 127], ex)
            return carry2
        lax.fori_loop(0, BLK // L, grp, 0)
        return carry
    lax.fori_loop(0, STAT_E // BLK, stat_block, 0)

    for k in range(NPAD // NS // CHUNK):
        pltpu.sync_copy(rows0, scores_sh.at[pl.ds(pl.multiple_of(s * (NPAD // NS) + k * CHUNK, 8), CHUNK)])

    @pl.when(s == 0)
    def _():
        pltpu.sync_copy(s_in_loc, s_in_sh)
        pltpu.sync_copy(s_out_loc, s_out_sh)
    plsc.subcore_barrier()

    @pl.when(s != 0)
    def _():
        pltpu.sync_copy(s_in_loc, s_in_sh.at[riota], add=True)
        pltpu.sync_copy(s_out_loc, s_out_sh.at[riota], add=True)
    plsc.subcore_barrier()

    pltpu.sync_copy(s_in_sh, s_in_loc)
    pltpu.sync_copy(s_out_sh, s_out_loc)

    def prop_block(blk, carry):
        base = pl.multiple_of(c * (EP // NC) + s * PROP_E + blk * BLK, BLK)
        rowb = pl.multiple_of(c * (EP // NC // CHUNK) + s * (PROP_E // CHUNK)
                              + blk * CPB, 8)
        pltpu.sync_copy(to2d_hbm.at[pl.ds(rowb, CPB)], to2d)
        pltpu.sync_copy(from_hbm.at[pl.ds(base, BLK)], fromb)
        pltpu.sync_copy(attr_hbm.at[pl.ds(base, BLK)], attrb)

        def do_chunk(jj, cur, oth, gsem_c, gsem_o):
            @pl.when(jj + 1 < CPB)
            def _():
                nsl = pl.ds((jj + 1) * CHUNK, CHUNK)
                pltpu.async_copy(upT_hbm.at[fromb.at[nsl]], oth, gsem_o)

            for k in range(CHUNK // L):
                sl = pl.ds(jj * CHUNK + k * L, L)
                ex = jnp.exp(attrb[sl])
                tt = to2d[jj, pl.ds(k * L, L)]
                ff = fromb[sl]
                g_in = plsc.load_gather(s_in_loc, [tt >> 7, tt & 127])
                g_out = plsc.load_gather(s_out_loc, [ff >> 7, ff & 127])
                y = (ex * ex) / (g_in * g_out) + 1e-10
                wbuf[pl.ds(k * L, L)] = _sqrt16(y)

            pltpu.make_async_copy(upT_hbm.at[pl.ds(0, CHUNK)], cur, gsem_c).wait()

            def scale_edge(e, carry3):
                wv = plsc.load_gather(wbuf, [jnp.full((L,), e, jnp.int32)])
                for v in range(B // L):
                    sl = pl.ds(v * L, L)
                    cur[e, sl] = cur[e, sl] * wv
                return carry3
            lax.fori_loop(0, CHUNK, scale_edge, 0)

            pltpu.sync_copy(cur, scores_sh.at[to2d.at[jj]], add=True)

        pltpu.async_copy(upT_hbm.at[fromb.at[pl.ds(0, CHUNK)]], rows0, gsem0)

        def chunk_body(jj, carry2):
            @pl.when(jj % 2 == 0)
            def _():
                do_chunk(jj, rows0, rows1, gsem0, gsem1)

            @pl.when(jj % 2 == 1)
            def _():
                do_chunk(jj, rows1, rows0, gsem1, gsem0)
            return carry2
        lax.fori_loop(0, CPB, chunk_body, 0)
        return carry
    lax.fori_loop(0, PROP_E // BLK, prop_block, 0)

    plsc.subcore_barrier()

    @pl.when(s < 10)
    def _():
        r0 = pl.multiple_of(s * 1000, 8)
        pltpu.sync_copy(scores_sh.at[pl.ds(r0, 1000)],
                        out_hbm.at[c, pl.ds(r0, 1000)])


@jax.jit
def _sc_spmm(from_, attrs, to2d, upT):
    mesh = plsc.VectorSubcoreMesh(core_axis_name="c", subcore_axis_name="s",
                                  num_cores=NC, num_subcores=NS)
    return pl.kernel(
        _sc_body,
        out_type=jax.ShapeDtypeStruct((NC, N, B), jnp.float32),
        mesh=mesh,
        compiler_params=pltpu.CompilerParams(needs_layout_passes=False),
        scratch_types=[
            pltpu.VMEM((BLK,), jnp.int32),
            pltpu.VMEM((BLK,), jnp.float32),
            pltpu.VMEM((CPB, CHUNK), jnp.int32),
            pltpu.VMEM((SROW, B), jnp.float32),
            pltpu.VMEM((SROW, B), jnp.float32),
            pltpu.VMEM((SROW,), jnp.int32),
            pltpu.VMEM((CHUNK,), jnp.float32),
            pltpu.VMEM((CHUNK, B), jnp.float32),
            pltpu.VMEM((CHUNK, B), jnp.float32),
            pltpu.SemaphoreType.DMA,
            pltpu.SemaphoreType.DMA,
            pltpu.VMEM_SHARED((SROW, B), jnp.float32),
            pltpu.VMEM_SHARED((SROW, B), jnp.float32),
            pltpu.VMEM_SHARED((NPAD, B), jnp.float32),
        ],
    )(from_, attrs, to2d, upT)


def _combine_body(p_ref, o_ref):
    o_ref[...] = (p_ref[0] + p_ref[1]).T


@jax.jit
def _tc_combine(partials):
    return pl.pallas_call(
        _combine_body,
        out_shape=jax.ShapeDtypeStruct((B, N), jnp.float32),
    )(partials)


def kernel(user_profiles, edge_attrs, edge_index):
    npad = EP - E
    pad_idx = N + jnp.arange(npad, dtype=jnp.int32) % NTRASH
    from_ = jnp.concatenate([edge_index[0], pad_idx])
    to_ = jnp.concatenate([edge_index[1], pad_idx])
    attrs = jnp.concatenate([edge_attrs, jnp.zeros((npad,), jnp.float32)])
    to2d = to_.reshape(EP // CHUNK, CHUNK)
    upT = jnp.pad(user_profiles.T, ((0, NPAD - N), (0, 0)))
    partials = _sc_spmm(from_, attrs, to2d, upT)
    return _tc_combine(partials)


# speedup vs baseline: 38.6117x; 1.0035x over previous
import jax
import jax.numpy as jnp
from jax import lax
from jax.experimental import pallas as pl
from jax.experimental.pallas import tpu as pltpu
from jax.experimental.pallas import tpu_sc as plsc

N = 10000
E = 320000
B = 128
NC = 2
NS = 16
L = 16

EP = 327680
NPAD = 10240
NTRASH = 224

STAT_E = EP // NS
PROP_E = EP // (NC * NS)
BLK = 2048
CHUNK = 64
CPB = BLK // CHUNK
SROW = NPAD // B


def _sqrt16(y):
    i = plsc.bitcast(y, jnp.int32)
    r = plsc.bitcast(jnp.int32(0x5F3759DF) - (i >> 1), jnp.float32)
    for _ in range(3):
        r = r * (1.5 - 0.5 * y * r * r)
    return y * r


def _sc_body(from_hbm, attr_hbm, to2d_hbm, upT_hbm, out_hbm,
             fromb, attrb, to2d, s_in_loc, s_out_loc, riota, wbuf,
             rows0, rows1, gsem0, gsem1, ssem0, ssem1,
             s_in_sh, s_out_sh, scores_sh):
    c = lax.axis_index("c")
    s = lax.axis_index("s")

    def zero_srow(r, carry):
        for g in range(B // L):
            s_in_loc[r, pl.ds(g * L, L)] = jnp.zeros((L,), jnp.float32)
            s_out_loc[r, pl.ds(g * L, L)] = jnp.zeros((L,), jnp.float32)
        return carry
    lax.fori_loop(0, SROW, zero_srow, 0)

    def zero_rows(r, carry):
        for g in range(B // L):
            rows0[r, pl.ds(g * L, L)] = jnp.zeros((L,), jnp.float32)
        return carry
    lax.fori_loop(0, CHUNK, zero_rows, 0)

    for r in range(SROW // L):
        riota[pl.ds(r * L, L)] = lax.iota(jnp.int32, L) + r * L

    def stat_block(blk, carry):
        base = pl.multiple_of(s * STAT_E + blk * BLK, BLK)
        rowb = pl.multiple_of(s * (STAT_E // CHUNK) + blk * CPB, 8)
        pltpu.sync_copy(to2d_hbm.at[pl.ds(rowb, CPB)], to2d)
        pltpu.sync_copy(from_hbm.at[pl.ds(base, BLK)], fromb)
        pltpu.sync_copy(attr_hbm.at[pl.ds(base, BLK)], attrb)

        def grp(g, carry2):
            r = g // (CHUNK // L)
            k = g - r * (CHUNK // L)
            ex = jnp.exp(attrb[pl.ds(g * L, L)])
            tt = to2d[r, pl.ds(k * L, L)]
            ff = fromb[pl.ds(g * L, L)]
            plsc.addupdate_scatter(s_in_loc, [tt >> 7, tt & 127], ex)
            plsc.addupdate_scatter(s_out_loc, [ff >> 7, ff & 127], ex)
            return carry2
        lax.fori_loop(0, BLK // L, grp, 0)
        return carry
    lax.fori_loop(0, STAT_E // BLK, stat_block, 0)

    for k in range(NPAD // NS // CHUNK):
        pltpu.sync_copy(rows0, scores_sh.at[pl.ds(pl.multiple_of(s * (NPAD // NS) + k * CHUNK, 8), CHUNK)])

    @pl.when(s == 0)
    def _():
        pltpu.sync_copy(s_in_loc, s_in_sh)
        pltpu.sync_copy(s_out_loc, s_out_sh)
    plsc.subcore_barrier()

    @pl.when(s != 0)
    def _():
        pltpu.sync_copy(s_in_loc, s_in_sh.at[riota], add=True)
        pltpu.sync_copy(s_out_loc, s_out_sh.at[riota], add=True)
    plsc.subcore_barrier()

    pltpu.sync_copy(s_in_sh, s_in_loc)
    pltpu.sync_copy(s_out_sh, s_out_loc)

    def prop_block(blk, carry):
        base = pl.multiple_of(c * (EP // NC) + s * PROP_E + blk * BLK, BLK)
        rowb = pl.multiple_of(c * (EP // NC // CHUNK) + s * (PROP_E // CHUNK)
                              + blk * CPB, 8)
        pltpu.sync_copy(to2d_hbm.at[pl.ds(rowb, CPB)], to2d)
        pltpu.sync_copy(from_hbm.at[pl.ds(base, BLK)], fromb)
        pltpu.sync_copy(attr_hbm.at[pl.ds(base, BLK)], attrb)

        def do_chunk(jj, cur, oth, gsem_c, gsem_o, ssem_c, ssem_o):
            @pl.when(jj >= 1)
            def _():
                pltpu.make_async_copy(upT_hbm.at[pl.ds(0, CHUNK)], oth,
                                      ssem_o).wait()

            @pl.when(jj + 1 < CPB)
            def _():
                nsl = pl.ds((jj + 1) * CHUNK, CHUNK)
                pltpu.async_copy(upT_hbm.at[fromb.at[nsl]], oth, gsem_o)

            for k in range(CHUNK // L):
                sl = pl.ds(jj * CHUNK + k * L, L)
                ex = jnp.exp(attrb[sl])
                tt = to2d[jj, pl.ds(k * L, L)]
                ff = fromb[sl]
                g_in = plsc.load_gather(s_in_loc, [tt >> 7, tt & 127])
                g_out = plsc.load_gather(s_out_loc, [ff >> 7, ff & 127])
                y = (ex * ex) / (g_in * g_out) + 1e-10
                wbuf[pl.ds(k * L, L)] = _sqrt16(y)

            pltpu.make_async_copy(upT_hbm.at[pl.ds(0, CHUNK)], cur, gsem_c).wait()

            def scale_edge(e, carry3):
                wv = plsc.load_gather(wbuf, [jnp.full((L,), e, jnp.int32)])
                for v in range(B // L):
                    sl = pl.ds(v * L, L)
                    cur[e, sl] = cur[e, sl] * wv
                return carry3
            lax.fori_loop(0, CHUNK, scale_edge, 0)

            pltpu.async_copy(cur, scores_sh.at[to2d.at[jj]], ssem_c,
                             add=True)

        pltpu.async_copy(upT_hbm.at[fromb.at[pl.ds(0, CHUNK)]], rows0, gsem0)

        def chunk_body(jj, carry2):
            @pl.when(jj % 2 == 0)
            def _():
                do_chunk(jj, rows0, rows1, gsem0, gsem1, ssem0, ssem1)

            @pl.when(jj % 2 == 1)
            def _():
                do_chunk(jj, rows1, rows0, gsem1, gsem0, ssem1, ssem0)
            return carry2
        lax.fori_loop(0, CPB, chunk_body, 0)

        pltpu.make_async_copy(upT_hbm.at[pl.ds(0, CHUNK)], rows1, ssem1).wait()
        return carry
    lax.fori_loop(0, PROP_E // BLK, prop_block, 0)

    plsc.subcore_barrier()

    @pl.when(s < 10)
    def _():
        r0 = pl.multiple_of(s * 1000, 8)
        pltpu.sync_copy(scores_sh.at[pl.ds(r0, 1000)],
                        out_hbm.at[c, pl.ds(r0, 1000)])


@jax.jit
def _sc_spmm(from_, attrs, to2d, upT):
    mesh = plsc.VectorSubcoreMesh(core_axis_name="c", subcore_axis_name="s",
                                  num_cores=NC, num_subcores=NS)
    return pl.kernel(
        _sc_body,
        out_type=jax.ShapeDtypeStruct((NC, N, B), jnp.float32),
        mesh=mesh,
        compiler_params=pltpu.CompilerParams(needs_layout_passes=False),
        scratch_types=[
            pltpu.VMEM((BLK,), jnp.int32),
            pltpu.VMEM((BLK,), jnp.float32),
            pltpu.VMEM((CPB, CHUNK), jnp.int32),
            pltpu.VMEM((SROW, B), jnp.float32),
            pltpu.VMEM((SROW, B), jnp.float32),
            pltpu.VMEM((SROW,), jnp.int32),
            pltpu.VMEM((CHUNK,), jnp.float32),
            pltpu.VMEM((CHUNK, B), jnp.float32),
            pltpu.VMEM((CHUNK, B), jnp.float32),
            pltpu.SemaphoreType.DMA,
            pltpu.SemaphoreType.DMA,
            pltpu.SemaphoreType.DMA,
            pltpu.SemaphoreType.DMA,
            pltpu.VMEM_SHARED((SROW, B), jnp.float32),
            pltpu.VMEM_SHARED((SROW, B), jnp.float32),
            pltpu.VMEM_SHARED((NPAD, B), jnp.float32),
        ],
    )(from_, attrs, to2d, upT)


def _combine_body(p_ref, o_ref):
    o_ref[...] = (p_ref[0] + p_ref[1]).T


@jax.jit
def _tc_combine(partials):
    return pl.pallas_call(
        _combine_body,
        out_shape=jax.ShapeDtypeStruct((B, N), jnp.float32),
    )(partials)


def kernel(user_profiles, edge_attrs, edge_index):
    npad = EP - E
    pad_idx = N + jnp.arange(npad, dtype=jnp.int32) % NTRASH
    from_ = jnp.concatenate([edge_index[0], pad_idx])
    to_ = jnp.concatenate([edge_index[1], pad_idx])
    attrs = jnp.concatenate([edge_attrs, jnp.zeros((npad,), jnp.float32)])
    to2d = to_.reshape(EP // CHUNK, CHUNK)
    upT = jnp.pad(user_profiles.T, ((0, NPAD - N), (0, 0)))
    partials = _sc_spmm(from_, attrs, to2d, upT)
    return _tc_combine(partials)


# speedup vs baseline: 45.7430x; 1.1847x over previous
import jax
import jax.numpy as jnp
from jax import lax
from jax.experimental import pallas as pl
from jax.experimental.pallas import tpu as pltpu
from jax.experimental.pallas import tpu_sc as plsc

N = 10000
E = 320000
B = 128
NC = 2
NS = 16
L = 16

EP = 327680
NPAD = 10240
NTRASH = 224

STAT_E = EP // NS
PROP_E = EP // (NC * NS)
BLK = 2048
CHUNK = 64
CPB = BLK // CHUNK
SROW = NPAD // B


def _sqrt16(y):
    i = plsc.bitcast(y, jnp.int32)
    r = plsc.bitcast(jnp.int32(0x5F3759DF) - (i >> 1), jnp.float32)
    for _ in range(3):
        r = r * (1.5 - 0.5 * y * r * r)
    return y * r


def _sc_body(from_hbm, attr_hbm, to2d_hbm, upT_hbm, out_hbm,
             fromb, attrb, to2d, s_in_loc, s_out_loc, riota, wbuf,
             rows0, rows1, gsem0, gsem1, ssem0, ssem1,
             s_in_sh, s_out_sh, scores_sh):
    c = lax.axis_index("c")
    s = lax.axis_index("s")

    def zero_srow(r, carry):
        for g in range(B // L):
            s_in_loc[r, pl.ds(g * L, L)] = jnp.zeros((L,), jnp.float32)
            s_out_loc[r, pl.ds(g * L, L)] = jnp.zeros((L,), jnp.float32)
        return carry
    lax.fori_loop(0, SROW, zero_srow, 0)

    def zero_rows(r, carry):
        for g in range(B // L):
            rows0[r, pl.ds(g * L, L)] = jnp.zeros((L,), jnp.float32)
        return carry
    lax.fori_loop(0, CHUNK, zero_rows, 0)

    for r in range(SROW // L):
        riota[pl.ds(r * L, L)] = lax.iota(jnp.int32, L) + r * L

    def stat_block(blk, carry):
        base = pl.multiple_of(s * STAT_E + blk * BLK, BLK)
        rowb = pl.multiple_of(s * (STAT_E // CHUNK) + blk * CPB, 8)
        pltpu.sync_copy(to2d_hbm.at[pl.ds(rowb, CPB)], to2d)
        pltpu.sync_copy(from_hbm.at[pl.ds(base, BLK)], fromb)
        pltpu.sync_copy(attr_hbm.at[pl.ds(base, BLK)], attrb)

        def grp(g, carry2):
            r = g // (CHUNK // L)
            k = g - r * (CHUNK // L)
            ex = jnp.exp(attrb[pl.ds(g * L, L)])
            tt = to2d[r, pl.ds(k * L, L)]
            ff = fromb[pl.ds(g * L, L)]
            plsc.addupdate_scatter(s_in_loc, [tt >> 7, tt & 127], ex)
            plsc.addupdate_scatter(s_out_loc, [ff >> 7, ff & 127], ex)
            return carry2
        lax.fori_loop(0, BLK // L, grp, 0)
        return carry
    lax.fori_loop(0, STAT_E // BLK, stat_block, 0)

    for k in range(NPAD // NS // CHUNK):
        pltpu.async_copy(rows0, scores_sh.at[pl.ds(pl.multiple_of(s * (NPAD // NS) + k * CHUNK, 8), CHUNK)], ssem0)

    @pl.when(s == 0)
    def _():
        pltpu.sync_copy(s_in_loc, s_in_sh)
        pltpu.sync_copy(s_out_loc, s_out_sh)
    plsc.subcore_barrier()

    @pl.when(s != 0)
    def _():
        pltpu.sync_copy(s_in_loc, s_in_sh.at[riota], add=True)
        pltpu.sync_copy(s_out_loc, s_out_sh.at[riota], add=True)

    for k in range(NPAD // NS // CHUNK):
        pltpu.make_async_copy(upT_hbm.at[pl.ds(0, CHUNK)], rows0, ssem0).wait()
    plsc.subcore_barrier()

    pltpu.sync_copy(s_in_sh, s_in_loc)
    pltpu.sync_copy(s_out_sh, s_out_loc)

    def prop_block(blk, carry):
        base = pl.multiple_of(c * (EP // NC) + s * PROP_E + blk * BLK, BLK)
        rowb = pl.multiple_of(c * (EP // NC // CHUNK) + s * (PROP_E // CHUNK)
                              + blk * CPB, 8)
        pltpu.sync_copy(to2d_hbm.at[pl.ds(rowb, CPB)], to2d)
        pltpu.sync_copy(from_hbm.at[pl.ds(base, BLK)], fromb)
        pltpu.sync_copy(attr_hbm.at[pl.ds(base, BLK)], attrb)

        def do_chunk(jj, cur, oth, gsem_c, gsem_o, ssem_c, ssem_o):
            @pl.when(jj >= 1)
            def _():
                pltpu.make_async_copy(upT_hbm.at[pl.ds(0, CHUNK)], oth,
                                      ssem_o).wait()

            @pl.when(jj + 1 < CPB)
            def _():
                nsl = pl.ds((jj + 1) * CHUNK, CHUNK)
                pltpu.async_copy(upT_hbm.at[fromb.at[nsl]], oth, gsem_o)

            for k in range(CHUNK // L):
                sl = pl.ds(jj * CHUNK + k * L, L)
                ex = jnp.exp(attrb[sl])
                tt = to2d[jj, pl.ds(k * L, L)]
                ff = fromb[sl]
                g_in = plsc.load_gather(s_in_loc, [tt >> 7, tt & 127])
                g_out = plsc.load_gather(s_out_loc, [ff >> 7, ff & 127])
                y = (ex * ex) / (g_in * g_out) + 1e-10
                wbuf[pl.ds(k * L, L)] = _sqrt16(y)

            pltpu.make_async_copy(upT_hbm.at[pl.ds(0, CHUNK)], cur, gsem_c).wait()

            @plsc.parallel_loop(0, CHUNK, 1, unroll=4)
            def _(e):
                wv = plsc.load_gather(wbuf, [jnp.full((L,), e, jnp.int32)])
                for v in range(B // L):
                    sl = pl.ds(v * L, L)
                    cur[e, sl] = cur[e, sl] * wv

            pltpu.async_copy(cur, scores_sh.at[to2d.at[jj]], ssem_c,
                             add=True)

        pltpu.async_copy(upT_hbm.at[fromb.at[pl.ds(0, CHUNK)]], rows0, gsem0)

        def chunk_body(jj, carry2):
            @pl.when(jj % 2 == 0)
            def _():
                do_chunk(jj, rows0, rows1, gsem0, gsem1, ssem0, ssem1)

            @pl.when(jj % 2 == 1)
            def _():
                do_chunk(jj, rows1, rows0, gsem1, gsem0, ssem1, ssem0)
            return carry2
        lax.fori_loop(0, CPB, chunk_body, 0)

        pltpu.make_async_copy(upT_hbm.at[pl.ds(0, CHUNK)], rows1, ssem1).wait()
        return carry
    lax.fori_loop(0, PROP_E // BLK, prop_block, 0)

    plsc.subcore_barrier()

    @pl.when(s < 10)
    def _():
        r0 = pl.multiple_of(s * 1000, 8)
        pltpu.sync_copy(scores_sh.at[pl.ds(r0, 1000)],
                        out_hbm.at[c, pl.ds(r0, 1000)])


@jax.jit
def _sc_spmm(from_, attrs, to2d, upT):
    mesh = plsc.VectorSubcoreMesh(core_axis_name="c", subcore_axis_name="s",
                                  num_cores=NC, num_subcores=NS)
    return pl.kernel(
        _sc_body,
        out_type=jax.ShapeDtypeStruct((NC, N, B), jnp.float32),
        mesh=mesh,
        compiler_params=pltpu.CompilerParams(needs_layout_passes=False),
        scratch_types=[
            pltpu.VMEM((BLK,), jnp.int32),
            pltpu.VMEM((BLK,), jnp.float32),
            pltpu.VMEM((CPB, CHUNK), jnp.int32),
            pltpu.VMEM((SROW, B), jnp.float32),
            pltpu.VMEM((SROW, B), jnp.float32),
            pltpu.VMEM((SROW,), jnp.int32),
            pltpu.VMEM((CHUNK,), jnp.float32),
            pltpu.VMEM((CHUNK, B), jnp.float32),
            pltpu.VMEM((CHUNK, B), jnp.float32),
            pltpu.SemaphoreType.DMA,
            pltpu.SemaphoreType.DMA,
            pltpu.SemaphoreType.DMA,
            pltpu.SemaphoreType.DMA,
            pltpu.VMEM_SHARED((SROW, B), jnp.float32),
            pltpu.VMEM_SHARED((SROW, B), jnp.float32),
            pltpu.VMEM_SHARED((NPAD, B), jnp.float32),
        ],
    )(from_, attrs, to2d, upT)


def _combine_body(p_ref, o_ref):
    o_ref[...] = (p_ref[0] + p_ref[1]).T


@jax.jit
def _tc_combine(partials):
    return pl.pallas_call(
        _combine_body,
        out_shape=jax.ShapeDtypeStruct((B, N), jnp.float32),
    )(partials)


def kernel(user_profiles, edge_attrs, edge_index):
    npad = EP - E
    pad_idx = N + jnp.arange(npad, dtype=jnp.int32) % NTRASH
    from_ = jnp.concatenate([edge_index[0], pad_idx])
    to_ = jnp.concatenate([edge_index[1], pad_idx])
    attrs = jnp.concatenate([edge_attrs, jnp.zeros((npad,), jnp.float32)])
    to2d = to_.reshape(EP // CHUNK, CHUNK)
    upT = jnp.pad(user_profiles.T, ((0, NPAD - N), (0, 0)))
    partials = _sc_spmm(from_, attrs, to2d, upT)
    return _tc_combine(partials)


# speedup vs baseline: 47.4980x; 1.0384x over previous
import jax
import jax.numpy as jnp
from jax import lax
from jax.experimental import pallas as pl
from jax.experimental.pallas import tpu as pltpu
from jax.experimental.pallas import tpu_sc as plsc

N = 10000
E = 320000
B = 128
NC = 2
NS = 16
L = 16

EP = 327680
NPAD = 10240
NTRASH = 224

STAT_E = EP // NS
PROP_E = EP // (NC * NS)
BLK = 2048
CHUNK = 64
CPB = BLK // CHUNK
SROW = NPAD // B


def _sqrt16(y):
    i = plsc.bitcast(y, jnp.int32)
    r = plsc.bitcast(jnp.int32(0x5F3759DF) - (i >> 1), jnp.float32)
    for _ in range(3):
        r = r * (1.5 - 0.5 * y * r * r)
    return y * r


def _sc_body(from_hbm, attr_hbm, to2d_hbm, upT_hbm, out_hbm,
             fromb, attrb, to2d, s_in_loc, s_out_loc, riota, wbuf,
             rows0, rows1, gsem0, gsem1, ssem0, ssem1,
             s_in_sh, s_out_sh, scores_sh):
    c = lax.axis_index("c")
    s = lax.axis_index("s")

    def zero_srow(r, carry):
        for g in range(B // L):
            s_in_loc[r, pl.ds(g * L, L)] = jnp.zeros((L,), jnp.float32)
            s_out_loc[r, pl.ds(g * L, L)] = jnp.zeros((L,), jnp.float32)
        return carry
    lax.fori_loop(0, SROW, zero_srow, 0)

    def zero_rows(r, carry):
        for g in range(B // L):
            rows0[r, pl.ds(g * L, L)] = jnp.zeros((L,), jnp.float32)
        return carry
    lax.fori_loop(0, CHUNK, zero_rows, 0)

    for r in range(SROW // L):
        riota[pl.ds(r * L, L)] = lax.iota(jnp.int32, L) + r * L

    def stat_block(blk, carry):
        base = pl.multiple_of(s * STAT_E + blk * BLK, BLK)
        rowb = pl.multiple_of(s * (STAT_E // CHUNK) + blk * CPB, 8)
        pltpu.sync_copy(to2d_hbm.at[pl.ds(rowb, CPB)], to2d)
        pltpu.sync_copy(from_hbm.at[pl.ds(base, BLK)], fromb)
        pltpu.sync_copy(attr_hbm.at[pl.ds(base, BLK)], attrb)

        @plsc.parallel_loop(0, BLK // L, 1, unroll=4)
        def _(g):
            r = g // (CHUNK // L)
            k = g - r * (CHUNK // L)
            ex = jnp.exp(attrb[pl.ds(g * L, L)])
            tt = to2d[r, pl.ds(k * L, L)]
            ff = fromb[pl.ds(g * L, L)]
            plsc.addupdate_scatter(s_in_loc, [tt >> 7, tt & 127], ex)
            plsc.addupdate_scatter(s_out_loc, [ff >> 7, ff & 127], ex)
        return carry
    lax.fori_loop(0, STAT_E // BLK, stat_block, 0)

    for k in range(NPAD // NS // CHUNK):
        pltpu.async_copy(rows0, scores_sh.at[pl.ds(pl.multiple_of(s * (NPAD // NS) + k * CHUNK, 8), CHUNK)], ssem0)

    @pl.when(s == 0)
    def _():
        pltpu.sync_copy(s_in_loc, s_in_sh)
        pltpu.sync_copy(s_out_loc, s_out_sh)
    plsc.subcore_barrier()

    @pl.when(s != 0)
    def _():
        pltpu.sync_copy(s_in_loc, s_in_sh.at[riota], add=True)
        pltpu.sync_copy(s_out_loc, s_out_sh.at[riota], add=True)

    for k in range(NPAD // NS // CHUNK):
        pltpu.make_async_copy(upT_hbm.at[pl.ds(0, CHUNK)], rows0, ssem0).wait()
    plsc.subcore_barrier()

    pltpu.sync_copy(s_in_sh, s_in_loc)
    pltpu.sync_copy(s_out_sh, s_out_loc)

    def prop_block(blk, carry):
        base = pl.multiple_of(c * (EP // NC) + s * PROP_E + blk * BLK, BLK)
        rowb = pl.multiple_of(c * (EP // NC // CHUNK) + s * (PROP_E // CHUNK)
                              + blk * CPB, 8)
        pltpu.sync_copy(to2d_hbm.at[pl.ds(rowb, CPB)], to2d)
        pltpu.sync_copy(from_hbm.at[pl.ds(base, BLK)], fromb)
        pltpu.sync_copy(attr_hbm.at[pl.ds(base, BLK)], attrb)

        def do_chunk(jj, cur, oth, gsem_c, gsem_o, ssem_c, ssem_o):
            @pl.when(jj >= 1)
            def _():
                pltpu.make_async_copy(upT_hbm.at[pl.ds(0, CHUNK)], oth,
                                      ssem_o).wait()

            @pl.when(jj + 1 < CPB)
            def _():
                nsl = pl.ds((jj + 1) * CHUNK, CHUNK)
                pltpu.async_copy(upT_hbm.at[fromb.at[nsl]], oth, gsem_o)

            for k in range(CHUNK // L):
                sl = pl.ds(jj * CHUNK + k * L, L)
                ex = jnp.exp(attrb[sl])
                tt = to2d[jj, pl.ds(k * L, L)]
                ff = fromb[sl]
                g_in = plsc.load_gather(s_in_loc, [tt >> 7, tt & 127])
                g_out = plsc.load_gather(s_out_loc, [ff >> 7, ff & 127])
                y = (ex * ex) / (g_in * g_out) + 1e-10
                wbuf[pl.ds(k * L, L)] = _sqrt16(y)

            pltpu.make_async_copy(upT_hbm.at[pl.ds(0, CHUNK)], cur, gsem_c).wait()

            @plsc.parallel_loop(0, CHUNK, 1, unroll=8)
            def _(e):
                wv = plsc.load_gather(wbuf, [jnp.full((L,), e, jnp.int32)])
                for v in range(B // L):
                    sl = pl.ds(v * L, L)
                    cur[e, sl] = cur[e, sl] * wv

            pltpu.async_copy(cur, scores_sh.at[to2d.at[jj]], ssem_c,
                             add=True)

        pltpu.async_copy(upT_hbm.at[fromb.at[pl.ds(0, CHUNK)]], rows0, gsem0)

        def chunk_body(jj, carry2):
            @pl.when(jj % 2 == 0)
            def _():
                do_chunk(jj, rows0, rows1, gsem0, gsem1, ssem0, ssem1)

            @pl.when(jj % 2 == 1)
            def _():
                do_chunk(jj, rows1, rows0, gsem1, gsem0, ssem1, ssem0)
            return carry2
        lax.fori_loop(0, CPB, chunk_body, 0)

        pltpu.make_async_copy(upT_hbm.at[pl.ds(0, CHUNK)], rows1, ssem1).wait()
        return carry
    lax.fori_loop(0, PROP_E // BLK, prop_block, 0)

    plsc.subcore_barrier()

    @pl.when(s < 10)
    def _():
        r0 = pl.multiple_of(s * 1000, 8)
        pltpu.sync_copy(scores_sh.at[pl.ds(r0, 1000)],
                        out_hbm.at[c, pl.ds(r0, 1000)])


@jax.jit
def _sc_spmm(from_, attrs, to2d, upT):
    mesh = plsc.VectorSubcoreMesh(core_axis_name="c", subcore_axis_name="s",
                                  num_cores=NC, num_subcores=NS)
    return pl.kernel(
        _sc_body,
        out_type=jax.ShapeDtypeStruct((NC, N, B), jnp.float32),
        mesh=mesh,
        compiler_params=pltpu.CompilerParams(needs_layout_passes=False),
        scratch_types=[
            pltpu.VMEM((BLK,), jnp.int32),
            pltpu.VMEM((BLK,), jnp.float32),
            pltpu.VMEM((CPB, CHUNK), jnp.int32),
            pltpu.VMEM((SROW, B), jnp.float32),
            pltpu.VMEM((SROW, B), jnp.float32),
            pltpu.VMEM((SROW,), jnp.int32),
            pltpu.VMEM((CHUNK,), jnp.float32),
            pltpu.VMEM((CHUNK, B), jnp.float32),
            pltpu.VMEM((CHUNK, B), jnp.float32),
            pltpu.SemaphoreType.DMA,
            pltpu.SemaphoreType.DMA,
            pltpu.SemaphoreType.DMA,
            pltpu.SemaphoreType.DMA,
            pltpu.VMEM_SHARED((SROW, B), jnp.float32),
            pltpu.VMEM_SHARED((SROW, B), jnp.float32),
            pltpu.VMEM_SHARED((NPAD, B), jnp.float32),
        ],
    )(from_, attrs, to2d, upT)


def _combine_body(p_ref, o_ref):
    o_ref[...] = (p_ref[0] + p_ref[1]).T


@jax.jit
def _tc_combine(partials):
    return pl.pallas_call(
        _combine_body,
        out_shape=jax.ShapeDtypeStruct((B, N), jnp.float32),
    )(partials)


def kernel(user_profiles, edge_attrs, edge_index):
    npad = EP - E
    pad_idx = N + jnp.arange(npad, dtype=jnp.int32) % NTRASH
    from_ = jnp.concatenate([edge_index[0], pad_idx])
    to_ = jnp.concatenate([edge_index[1], pad_idx])
    attrs = jnp.concatenate([edge_attrs, jnp.zeros((npad,), jnp.float32)])
    to2d = to_.reshape(EP // CHUNK, CHUNK)
    upT = jnp.pad(user_profiles.T, ((0, NPAD - N), (0, 0)))
    partials = _sc_spmm(from_, attrs, to2d, upT)
    return _tc_combine(partials)


# speedup vs baseline: 47.6119x; 1.0024x over previous
import jax
import jax.numpy as jnp
from jax import lax
from jax.experimental import pallas as pl
from jax.experimental.pallas import tpu as pltpu
from jax.experimental.pallas import tpu_sc as plsc

N = 10000
E = 320000
B = 128
NC = 2
NS = 16
L = 16

EP = 327680
NPAD = 10240
NTRASH = 224

STAT_E = EP // NS
PROP_E = EP // (NC * NS)
BLK = 2048
CHUNK = 64
CPB = BLK // CHUNK
SROW = NPAD // B


def _sqrt16(y):
    i = plsc.bitcast(y, jnp.int32)
    r = plsc.bitcast(jnp.int32(0x5F3759DF) - (i >> 1), jnp.float32)
    for _ in range(3):
        r = r * (1.5 - 0.5 * y * r * r)
    return y * r


def _sc_body(from_hbm, attr_hbm, to2d_hbm, upT_hbm, out_hbm,
             fromb, attrb, to2d, s_in_loc, s_out_loc, riota, wbuf,
             rows0, rows1, gsem0, gsem1, ssem0, ssem1,
             s_in_sh, s_out_sh, scores_sh):
    c = lax.axis_index("c")
    s = lax.axis_index("s")

    def zero_srow(r, carry):
        for g in range(B // L):
            s_in_loc[r, pl.ds(g * L, L)] = jnp.zeros((L,), jnp.float32)
            s_out_loc[r, pl.ds(g * L, L)] = jnp.zeros((L,), jnp.float32)
        return carry
    lax.fori_loop(0, SROW, zero_srow, 0)

    def zero_rows(r, carry):
        for g in range(B // L):
            rows0[r, pl.ds(g * L, L)] = jnp.zeros((L,), jnp.float32)
        return carry
    lax.fori_loop(0, CHUNK, zero_rows, 0)

    for r in range(SROW // L):
        riota[pl.ds(r * L, L)] = lax.iota(jnp.int32, L) + r * L

    def stat_block(blk, carry):
        base = pl.multiple_of(s * STAT_E + blk * BLK, BLK)
        rowb = pl.multiple_of(s * (STAT_E // CHUNK) + blk * CPB, 8)
        pltpu.sync_copy(to2d_hbm.at[pl.ds(rowb, CPB)], to2d)
        pltpu.sync_copy(from_hbm.at[pl.ds(base, BLK)], fromb)
        pltpu.sync_copy(attr_hbm.at[pl.ds(base, BLK)], attrb)

        @plsc.parallel_loop(0, BLK // L, 1, unroll=4)
        def _(g):
            r = g // (CHUNK // L)
            k = g - r * (CHUNK // L)
            ex = jnp.exp(attrb[pl.ds(g * L, L)])
            tt = to2d[r, pl.ds(k * L, L)]
            ff = fromb[pl.ds(g * L, L)]
            plsc.addupdate_scatter(s_in_loc, [tt >> 7, tt & 127], ex)
            plsc.addupdate_scatter(s_out_loc, [ff >> 7, ff & 127], ex)
        return carry
    lax.fori_loop(0, STAT_E // BLK, stat_block, 0)

    for k in range(NPAD // NS // CHUNK):
        pltpu.async_copy(rows0, scores_sh.at[pl.ds(pl.multiple_of(s * (NPAD // NS) + k * CHUNK, 8), CHUNK)], ssem0)

    @pl.when(s == 0)
    def _():
        pltpu.sync_copy(s_in_loc, s_in_sh)
        pltpu.sync_copy(s_out_loc, s_out_sh)
    plsc.subcore_barrier()

    @pl.when(s != 0)
    def _():
        pltpu.sync_copy(s_in_loc, s_in_sh.at[riota], add=True)
        pltpu.sync_copy(s_out_loc, s_out_sh.at[riota], add=True)

    for k in range(NPAD // NS // CHUNK):
        pltpu.make_async_copy(upT_hbm.at[pl.ds(0, CHUNK)], rows0, ssem0).wait()
    plsc.subcore_barrier()

    pltpu.sync_copy(s_in_sh, s_in_loc)
    pltpu.sync_copy(s_out_sh, s_out_loc)

    def prop_block(blk, carry):
        base = pl.multiple_of(c * (EP // NC) + s * PROP_E + blk * BLK, BLK)
        rowb = pl.multiple_of(c * (EP // NC // CHUNK) + s * (PROP_E // CHUNK)
                              + blk * CPB, 8)
        pltpu.sync_copy(to2d_hbm.at[pl.ds(rowb, CPB)], to2d)
        pltpu.sync_copy(from_hbm.at[pl.ds(base, BLK)], fromb)
        pltpu.sync_copy(attr_hbm.at[pl.ds(base, BLK)], attrb)

        def do_chunk(jj, cur, oth, gsem_c, gsem_o, ssem_c, ssem_o):
            @pl.when(jj >= 1)
            def _():
                pltpu.make_async_copy(upT_hbm.at[pl.ds(0, CHUNK)], oth,
                                      ssem_o).wait()

            @pl.when(jj + 1 < CPB)
            def _():
                nsl = pl.ds((jj + 1) * CHUNK, CHUNK)
                pltpu.async_copy(upT_hbm.at[fromb.at[nsl]], oth, gsem_o)

            for k in range(CHUNK // L):
                sl = pl.ds(jj * CHUNK + k * L, L)
                ex = jnp.exp(attrb[sl])
                tt = to2d[jj, pl.ds(k * L, L)]
                ff = fromb[sl]
                g_in = plsc.load_gather(s_in_loc, [tt >> 7, tt & 127])
                g_out = plsc.load_gather(s_out_loc, [ff >> 7, ff & 127])
                y = (ex * ex) / (g_in * g_out) + 1e-10
                wbuf[pl.ds(k * L, L)] = _sqrt16(y)

            pltpu.make_async_copy(upT_hbm.at[pl.ds(0, CHUNK)], cur, gsem_c).wait()

            @plsc.parallel_loop(0, CHUNK, 1, unroll=8)
            def _(e):
                wv = plsc.load_gather(wbuf, [jnp.full((L,), e, jnp.int32)])
                for v in range(B // L):
                    sl = pl.ds(v * L, L)
                    cur[e, sl] = cur[e, sl] * wv

            pltpu.async_copy(cur, scores_sh.at[to2d.at[jj]], ssem_c,
                             add=True)

        pltpu.async_copy(upT_hbm.at[fromb.at[pl.ds(0, CHUNK)]], rows0, gsem0)

        def chunk_body(m, carry2):
            do_chunk(2 * m, rows0, rows1, gsem0, gsem1, ssem0, ssem1)
            do_chunk(2 * m + 1, rows1, rows0, gsem1, gsem0, ssem1, ssem0)
            return carry2
        lax.fori_loop(0, CPB // 2, chunk_body, 0)

        pltpu.make_async_copy(upT_hbm.at[pl.ds(0, CHUNK)], rows1, ssem1).wait()
        return carry
    lax.fori_loop(0, PROP_E // BLK, prop_block, 0)

    plsc.subcore_barrier()

    r0 = pl.multiple_of(s * (NPAD // NS), 8)
    pltpu.sync_copy(scores_sh.at[pl.ds(r0, NPAD // NS)],
                    out_hbm.at[c, pl.ds(r0, NPAD // NS)])


@jax.jit
def _sc_spmm(from_, attrs, to2d, upT):
    mesh = plsc.VectorSubcoreMesh(core_axis_name="c", subcore_axis_name="s",
                                  num_cores=NC, num_subcores=NS)
    return pl.kernel(
        _sc_body,
        out_type=jax.ShapeDtypeStruct((NC, NPAD, B), jnp.float32),
        mesh=mesh,
        compiler_params=pltpu.CompilerParams(needs_layout_passes=False),
        scratch_types=[
            pltpu.VMEM((BLK,), jnp.int32),
            pltpu.VMEM((BLK,), jnp.float32),
            pltpu.VMEM((CPB, CHUNK), jnp.int32),
            pltpu.VMEM((SROW, B), jnp.float32),
            pltpu.VMEM((SROW, B), jnp.float32),
            pltpu.VMEM((SROW,), jnp.int32),
            pltpu.VMEM((CHUNK,), jnp.float32),
            pltpu.VMEM((CHUNK, B), jnp.float32),
            pltpu.VMEM((CHUNK, B), jnp.float32),
            pltpu.SemaphoreType.DMA,
            pltpu.SemaphoreType.DMA,
            pltpu.SemaphoreType.DMA,
            pltpu.SemaphoreType.DMA,
            pltpu.VMEM_SHARED((SROW, B), jnp.float32),
            pltpu.VMEM_SHARED((SROW, B), jnp.float32),
            pltpu.VMEM_SHARED((NPAD, B), jnp.float32),
        ],
    )(from_, attrs, to2d, upT)


def _combine_body(p_ref, o_ref):
    o_ref[...] = (p_ref[0, :N] + p_ref[1, :N]).T


@jax.jit
def _tc_combine(partials):
    return pl.pallas_call(
        _combine_body,
        out_shape=jax.ShapeDtypeStruct((B, N), jnp.float32),
    )(partials)


def kernel(user_profiles, edge_attrs, edge_index):
    npad = EP - E
    pad_idx = N + jnp.arange(npad, dtype=jnp.int32) % NTRASH
    from_ = jnp.concatenate([edge_index[0], pad_idx])
    to_ = jnp.concatenate([edge_index[1], pad_idx])
    attrs = jnp.concatenate([edge_attrs, jnp.zeros((npad,), jnp.float32)])
    to2d = to_.reshape(EP // CHUNK, CHUNK)
    upT = jnp.pad(user_profiles.T, ((0, NPAD - N), (0, 0)))
    partials = _sc_spmm(from_, attrs, to2d, upT)
    return _tc_combine(partials)
